# baseline (speedup 1.0000x reference)
import functools

import jax
import jax.numpy as jnp
from jax import lax
from jax.experimental import pallas as pl
from jax.experimental.pallas import tpu as pltpu
from jax.experimental.pallas import tpu_sc as plsc

BS = 512


def _ln_kernel(x_ref, p_ref, w_ref, b_ref, o_ref):
    e = x_ref[...] + p_ref[None]
    mean = jnp.mean(e, axis=-1, keepdims=True)
    c = e - mean
    var = jnp.mean(c * c, axis=-1, keepdims=True)
    inv = jax.lax.rsqrt(var + 1e-5)
    o_ref[...] = (c * inv) * w_ref[0] + b_ref[0]


@jax.jit
def _run_tc(x, pos_table, ln_w, ln_b):
    B, S, D = x.shape
    grid = (S // BS,)
    return pl.pallas_call(
        _ln_kernel,
        grid=grid,
        in_specs=[
            pl.BlockSpec((B, BS, D), lambda s: (0, s, 0)),
            pl.BlockSpec((BS, D), lambda s: (s, 0)),
            pl.BlockSpec((1, D), lambda s: (0, 0)),
            pl.BlockSpec((1, D), lambda s: (0, 0)),
        ],
        out_specs=pl.BlockSpec((B, BS, D), lambda s: (0, s, 0)),
        out_shape=jax.ShapeDtypeStruct((B, S, D), x.dtype),
        compiler_params=pltpu.CompilerParams(
            dimension_semantics=("arbitrary",),
        ),
    )(x, pos_table, ln_w.reshape(1, D), ln_b.reshape(1, D))


_B, _S, _D = 4, 8192, 1024
_NC, _NS = 2, 16
_NW = _NC * _NS
_SEG = (_B * _S) // _NW
_T = 16
_NV = _D // 16


def _lane_sum(v):
    lanes = lax.iota(jnp.int32, 16)
    for sh in (8, 4, 2, 1):
        perm = lanes ^ sh
        v = v + lax.gather(
            v, perm[:, None],
            dimension_numbers=lax.GatherDimensionNumbers(
                offset_dims=(), collapsed_slice_dims=(0,),
                start_index_map=(0,)),
            slice_sizes=(1,),
            mode=lax.GatherScatterMode.PROMISE_IN_BOUNDS)
    return v


def _sc_body(x_hbm, pos_hbm, w_hbm, b_hbm, out_hbm, xt, ptile, wt, bt):
    wid = lax.axis_index("s") * _NC + lax.axis_index("c")
    per_b = _NW // _B
    bidx = wid // per_b
    row0 = (wid % per_b) * _SEG
    pltpu.sync_copy(w_hbm, wt)
    pltpu.sync_copy(b_hbm, bt)

    def tile_body(g, _):
        r0 = row0 + g * _T
        pltpu.sync_copy(x_hbm.at[bidx, pl.ds(r0, _T), :], xt)
        pltpu.sync_copy(pos_hbm.at[pl.ds(r0, _T), :], ptile)
        for r in range(_T):
            def acc_body(i, carry):
                sv, qv = carry
                sl = pl.ds(i * 16, 16)
                v = xt[r, sl] + ptile[r, sl]
                xt[r, sl] = v
                return sv + v, qv + v * v

            sv, qv = lax.fori_loop(
                0, _NV, acc_body,
                (jnp.zeros((16,), jnp.float32), jnp.zeros((16,), jnp.float32)))
            mean = _lane_sum(sv) * (1.0 / _D)
            var = _lane_sum(qv) * (1.0 / _D) - mean * mean
            xv = var + 1e-5
            bits = lax.bitcast_convert_type(xv, jnp.int32)
            y = lax.bitcast_convert_type(
                jnp.int32(0x5F3759DF) - (bits >> 1), jnp.float32)
            for _i in range(4):
                y = y * (1.5 - 0.5 * xv * y * y)

            def norm_body(i, c):
                sl = pl.ds(i * 16, 16)
                e = xt[r, sl]
                xt[r, sl] = (e - mean) * y * wt[sl] + bt[sl]
                return c

            lax.fori_loop(0, _NV, norm_body, 0)
        pltpu.sync_copy(xt, out_hbm.at[bidx, pl.ds(r0, _T), :])
        return _

    lax.fori_loop(0, _SEG // _T, tile_body, 0)


@jax.jit
def _run_sc(x, pos_table, ln_w, ln_b):
    mesh = plsc.VectorSubcoreMesh(core_axis_name="c", subcore_axis_name="s")
    f = functools.partial(
        pl.kernel,
        mesh=mesh,
        out_type=jax.ShapeDtypeStruct((_B, _S, _D), jnp.float32),
        scratch_types=[
            pltpu.VMEM((_T, _D), jnp.float32),
            pltpu.VMEM((_T, _D), jnp.float32),
            pltpu.VMEM((_D,), jnp.float32),
            pltpu.VMEM((_D,), jnp.float32),
        ],
    )(_sc_body)
    return f(x, pos_table, ln_w, ln_b)


def kernel(x, batch_size, pos_table, ln_w, ln_b):
    return _run_sc(x, pos_table, ln_w, ln_b)


# speedup vs baseline: 12.1957x; 12.1957x over previous
import functools

import jax
import jax.numpy as jnp
from jax import lax
from jax.experimental import pallas as pl
from jax.experimental.pallas import tpu as pltpu
from jax.experimental.pallas import tpu_sc as plsc

BS = 512


def _ln_kernel(x_ref, p_ref, w_ref, b_ref, o_ref):
    e = x_ref[...] + p_ref[None]
    mean = jnp.mean(e, axis=-1, keepdims=True)
    c = e - mean
    var = jnp.mean(c * c, axis=-1, keepdims=True)
    inv = jax.lax.rsqrt(var + 1e-5)
    o_ref[...] = (c * inv) * w_ref[0] + b_ref[0]


@jax.jit
def _run_tc(x, pos_table, ln_w, ln_b):
    B, S, D = x.shape
    grid = (S // BS,)
    return pl.pallas_call(
        _ln_kernel,
        grid=grid,
        in_specs=[
            pl.BlockSpec((B, BS, D), lambda s: (0, s, 0)),
            pl.BlockSpec((BS, D), lambda s: (s, 0)),
            pl.BlockSpec((1, D), lambda s: (0, 0)),
            pl.BlockSpec((1, D), lambda s: (0, 0)),
        ],
        out_specs=pl.BlockSpec((B, BS, D), lambda s: (0, s, 0)),
        out_shape=jax.ShapeDtypeStruct((B, S, D), x.dtype),
        compiler_params=pltpu.CompilerParams(
            dimension_semantics=("arbitrary",),
        ),
    )(x, pos_table, ln_w.reshape(1, D), ln_b.reshape(1, D))


_B, _S, _D = 4, 8192, 1024
_NC, _NS = 2, 16
_NW = _NC * _NS
_SEG = (_B * _S) // _NW
_T = 16
_NV = _D // 16


def _lane_sum(v):
    lanes = lax.iota(jnp.int32, 16)
    for sh in (8, 4, 2, 1):
        perm = lanes ^ sh
        v = v + lax.gather(
            v, perm[:, None],
            dimension_numbers=lax.GatherDimensionNumbers(
                offset_dims=(), collapsed_slice_dims=(0,),
                start_index_map=(0,)),
            slice_sizes=(1,),
            mode=lax.GatherScatterMode.PROMISE_IN_BOUNDS)
    return v


def _sc_body(x_hbm, pos_hbm, w_hbm, b_hbm, out_hbm, xt, ptile, wt, bt):
    wid = lax.axis_index("s") * _NC + lax.axis_index("c")
    per_b = _NW // _B
    bidx = wid // per_b
    row0 = (wid % per_b) * _SEG
    pltpu.sync_copy(w_hbm, wt)
    pltpu.sync_copy(b_hbm, bt)

    def tile_body(g, _):
        r0 = row0 + g * _T
        pltpu.sync_copy(x_hbm.at[bidx, pl.ds(r0, _T), :], xt)
        pltpu.sync_copy(pos_hbm.at[pl.ds(r0, _T), :], ptile)
        for r in range(_T):
            def acc_body(i, carry):
                sv, qv = carry
                sl = pl.ds(i * 16, 16)
                v = xt[r, sl] + ptile[r, sl]
                xt[r, sl] = v
                return sv + v, qv + v * v

            sv, qv = lax.fori_loop(
                0, _NV, acc_body,
                (jnp.zeros((16,), jnp.float32), jnp.zeros((16,), jnp.float32)))
            mean = _lane_sum(sv) * (1.0 / _D)
            var = _lane_sum(qv) * (1.0 / _D) - mean * mean
            xv = var + 1e-5
            bits = lax.bitcast_convert_type(xv, jnp.int32)
            y = lax.bitcast_convert_type(
                jnp.int32(0x5F3759DF) - (bits >> 1), jnp.float32)
            for _i in range(4):
                y = y * (1.5 - 0.5 * xv * y * y)

            def norm_body(i, c):
                sl = pl.ds(i * 16, 16)
                e = xt[r, sl]
                xt[r, sl] = (e - mean) * y * wt[sl] + bt[sl]
                return c

            lax.fori_loop(0, _NV, norm_body, 0)
        pltpu.sync_copy(xt, out_hbm.at[bidx, pl.ds(r0, _T), :])
        return _

    lax.fori_loop(0, _SEG // _T, tile_body, 0)


@jax.jit
def _run_sc(x, pos_table, ln_w, ln_b):
    mesh = plsc.VectorSubcoreMesh(core_axis_name="c", subcore_axis_name="s")
    f = functools.partial(
        pl.kernel,
        mesh=mesh,
        out_type=jax.ShapeDtypeStruct((_B, _S, _D), jnp.float32),
        scratch_types=[
            pltpu.VMEM((_T, _D), jnp.float32),
            pltpu.VMEM((_T, _D), jnp.float32),
            pltpu.VMEM((_D,), jnp.float32),
            pltpu.VMEM((_D,), jnp.float32),
        ],
    )(_sc_body)
    return f(x, pos_table, ln_w, ln_b)


def kernel(x, batch_size, pos_table, ln_w, ln_b):
    return _run_tc(x, pos_table, ln_w, ln_b)
